# Initial kernel scaffold; baseline (speedup 1.0000x reference)
import jax
import jax.numpy as jnp
from jax.experimental import pallas as pl


def kernel(x, w1, b1, w2, g2, be2, w3, g3, be3, w4, g4, be4, w5, b5):
    raise NotImplementedError("write your pallas kernel here")


# speedup vs baseline: 37.4216x; 37.4216x over previous
import functools

import jax
import jax.numpy as jnp
from jax.experimental import pallas as pl
from jax.experimental.pallas import tpu as pltpu

LEAKY_SLOPE = 0.2
BN_EPS = 1e-5


def _s2d(h):
    N, H, _, C = h.shape
    Z = H // 2 + 1
    hp = jnp.pad(h, ((0, 0), (1, 1), (1, 1), (0, 0)))
    return (hp.reshape(N, Z, 2, Z, 2, C)
            .transpose(0, 1, 3, 2, 4, 5)
            .reshape(N, Z, Z, 4 * C))


def _wmat(w, C):
    Co, Ci = w.shape[0], w.shape[1]
    Wt = jnp.transpose(w, (2, 3, 1, 0)).astype(jnp.float32)
    Wt = jnp.pad(Wt, ((0, 0), (0, 0), (0, C - Ci), (0, 0)))
    Wt = Wt.reshape(2, 2, 2, 2, C, Co).transpose(0, 2, 1, 3, 4, 5)
    return Wt.reshape(4, 4 * C, Co)


def _bn_coeffs(stats, m_rows, g, be):
    st = stats.reshape(-1, 8, stats.shape[-1])
    ssum = jnp.sum(st[:, 0, :], axis=0)
    ssq = jnp.sum(st[:, 1, :], axis=0)
    mean = ssum / m_rows
    var = jnp.maximum(ssq / m_rows - mean * mean, 0.0)
    s = g * jax.lax.rsqrt(var + BN_EPS)
    t = be - mean * s
    v = -t / jnp.where(s == 0.0, 1.0, s)
    return s, t, v


def _taps_matmul(zb, b_ref, Ho):
    NB, _, _, K4 = zb.shape
    acc = None
    for t, (di, dj) in enumerate(((0, 0), (0, 1), (1, 0), (1, 1))):
        a = zb[:, di:di + Ho, dj:dj + Ho, :].reshape(NB * Ho * Ho, K4)
        d = jnp.dot(a, b_ref[t], preferred_element_type=jnp.float32)
        acc = d if acc is None else acc + d
    return acc


def _l1_kernel(z_ref, b_ref, bias_ref, o_ref, *, Ho):
    y = _taps_matmul(z_ref[...], b_ref, Ho) + bias_ref[...]
    o_ref[...] = jnp.where(y > 0, y, LEAKY_SLOPE * y)


def _conv_stats_kernel(z_ref, b_ref, o_ref, st_ref, *, Ho):
    acc = _taps_matmul(z_ref[...], b_ref, Ho)
    o_ref[...] = acc
    st_ref[0:1, :] = jnp.sum(acc, axis=0, keepdims=True)
    st_ref[1:2, :] = jnp.sum(acc * acc, axis=0, keepdims=True)


def _affine_conv_stats_kernel(z_ref, b_ref, s_ref, t_ref, o_ref, st_ref, *, Ho):
    zb = z_ref[...] * s_ref[...] + t_ref[...]
    zb = jnp.where(zb > 0, zb, LEAKY_SLOPE * zb)
    acc = _taps_matmul(zb, b_ref, Ho)
    o_ref[...] = acc
    st_ref[0:1, :] = jnp.sum(acc, axis=0, keepdims=True)
    st_ref[1:2, :] = jnp.sum(acc * acc, axis=0, keepdims=True)


def _l5_kernel(a_ref, b_ref, s_ref, t_ref, bias_ref, o_ref, acc_ref):
    k = pl.program_id(1)

    @pl.when(k == 0)
    def _():
        acc_ref[...] = jnp.zeros_like(acc_ref)

    z = a_ref[...] * s_ref[...] + t_ref[...]
    z = jnp.where(z > 0, z, LEAKY_SLOPE * z)
    acc_ref[...] += jnp.dot(z, b_ref[...], preferred_element_type=jnp.float32)

    @pl.when(k == pl.num_programs(1) - 1)
    def _():
        y = acc_ref[...] + bias_ref[...]
        o_ref[...] = 1.0 / (1.0 + jnp.exp(-y))


def _conv_layer(z, b_taps, Ho, Co, nb, affine=None):
    N, Z, _, K4 = z.shape
    grid = N // nb
    mb = nb * Ho * Ho
    in_specs = [
        pl.BlockSpec((nb, Z, Z, K4), lambda m: (m, 0, 0, 0)),
        pl.BlockSpec((4, K4, Co), lambda m: (0, 0, 0)),
    ]
    if affine is None:
        body = functools.partial(_conv_stats_kernel, Ho=Ho)
        args = (z, b_taps)
    else:
        s, t = affine
        body = functools.partial(_affine_conv_stats_kernel, Ho=Ho)
        in_specs += [pl.BlockSpec((1, K4), lambda m: (0, 0)),
                     pl.BlockSpec((1, K4), lambda m: (0, 0))]
        args = (z, b_taps, s.reshape(1, K4), t.reshape(1, K4))
    raw, stats = pl.pallas_call(
        body,
        out_shape=(jax.ShapeDtypeStruct((N * Ho * Ho, Co), jnp.float32),
                   jax.ShapeDtypeStruct((grid * 8, Co), jnp.float32)),
        grid=(grid,),
        in_specs=in_specs,
        out_specs=(pl.BlockSpec((mb, Co), lambda m: (m, 0)),
                   pl.BlockSpec((8, Co), lambda m: (m, 0))),
        compiler_params=pltpu.CompilerParams(
            dimension_semantics=("parallel",)),
    )(*args)
    return raw, stats


def kernel(x, w1, b1, w2, g2, be2, w3, g3, be3, w4, g4, be4, w5, b5):
    N = x.shape[0]
    G = N // 4

    xp = jnp.pad(x.astype(jnp.float32), ((0, 0), (0, 5), (1, 1), (1, 1)))
    z1 = (xp.reshape(G, 4, 8, 17, 2, 17, 2)
          .transpose(0, 3, 5, 1, 4, 6, 2)
          .reshape(G, 17, 17, 128))
    b32 = _wmat(w1, 8)
    eye4 = jnp.eye(4, dtype=jnp.float32)
    b1bd = jax.vmap(lambda b: jnp.kron(eye4, b))(b32)
    bias1 = jnp.tile(b1.astype(jnp.float32), 4).reshape(1, 256)

    nb1 = min(16, G)
    h1g = pl.pallas_call(
        functools.partial(_l1_kernel, Ho=16),
        out_shape=jax.ShapeDtypeStruct((G * 256, 256), jnp.float32),
        grid=(G // nb1,),
        in_specs=[
            pl.BlockSpec((nb1, 17, 17, 128), lambda m: (m, 0, 0, 0)),
            pl.BlockSpec((4, 128, 256), lambda m: (0, 0, 0)),
            pl.BlockSpec((1, 256), lambda m: (0, 0)),
        ],
        out_specs=pl.BlockSpec((nb1 * 256, 256), lambda m: (m, 0)),
        compiler_params=pltpu.CompilerParams(
            dimension_semantics=("parallel",)),
    )(z1, b1bd, bias1)

    h1gp = jnp.pad(h1g.reshape(G, 16, 16, 256),
                   ((0, 0), (1, 1), (1, 1), (0, 0)))
    z2 = (h1gp.reshape(G, 9, 2, 9, 2, 4, 64)
          .transpose(0, 5, 1, 3, 2, 4, 6)
          .reshape(N, 9, 9, 256))

    raw2, st2 = _conv_layer(z2, _wmat(w2, 64), 8, 128, nb=min(64, N))
    s2, t2, v2 = _bn_coeffs(st2, N * 64, g2, be2)

    z3 = _s2d(raw2.reshape(N, 8, 8, 128) - v2) + jnp.tile(v2, 4)
    raw3, st3 = _conv_layer(z3, _wmat(w3, 128), 4, 256, nb=min(128, N),
                            affine=(jnp.tile(s2, 4), jnp.tile(t2, 4)))
    s3, t3, v3 = _bn_coeffs(st3, N * 16, g3, be3)

    z4 = _s2d(raw3.reshape(N, 4, 4, 256) - v3) + jnp.tile(v3, 4)
    raw4, st4 = _conv_layer(z4, _wmat(w4, 256), 2, 512, nb=min(128, N),
                            affine=(jnp.tile(s3, 4), jnp.tile(t3, 4)))
    s4, t4, v4 = _bn_coeffs(st4, N * 4, g4, be4)

    z5 = (_s2d(raw4.reshape(N, 2, 2, 512) - v4) + jnp.tile(v4, 4)
          ).reshape(N, 8192)
    b5m = jnp.pad(_wmat(w5, 512).reshape(8192, 1), ((0, 0), (0, 127)))
    bias5 = jnp.pad(b5.astype(jnp.float32), (0, 127)).reshape(1, 128)
    s4z = jnp.tile(s4, 16).reshape(1, 8192)
    t4z = jnp.tile(t4, 16).reshape(1, 8192)

    nb5 = N // 2
    y = pl.pallas_call(
        _l5_kernel,
        out_shape=jax.ShapeDtypeStruct((N, 128), jnp.float32),
        grid=(2, 4),
        in_specs=[
            pl.BlockSpec((nb5, 2048), lambda m, k: (m, k)),
            pl.BlockSpec((2048, 128), lambda m, k: (k, 0)),
            pl.BlockSpec((1, 2048), lambda m, k: (0, k)),
            pl.BlockSpec((1, 2048), lambda m, k: (0, k)),
            pl.BlockSpec((1, 128), lambda m, k: (0, 0)),
        ],
        out_specs=pl.BlockSpec((nb5, 128), lambda m, k: (m, 0)),
        scratch_shapes=[pltpu.VMEM((nb5, 128), jnp.float32)],
        compiler_params=pltpu.CompilerParams(
            dimension_semantics=("parallel", "arbitrary")),
    )(z5, b5m, s4z, t4z, bias5)

    return y[:, :1].reshape(N, 1, 1, 1)


# speedup vs baseline: 55.2466x; 1.4763x over previous
import functools

import jax
import jax.numpy as jnp
from jax.experimental import pallas as pl
from jax.experimental.pallas import tpu as pltpu

LEAKY_SLOPE = 0.2
BN_EPS = 1e-5


def _wmat(w, C):
    Co, Ci = w.shape[0], w.shape[1]
    Wt = jnp.transpose(w, (2, 3, 1, 0)).astype(jnp.float32)
    Wt = jnp.pad(Wt, ((0, 0), (0, 0), (0, C - Ci), (0, 0)))
    Wt = Wt.reshape(2, 2, 2, 2, C, Co).transpose(0, 2, 1, 3, 4, 5)
    return Wt.reshape(4, 4 * C, Co)


def _bn_coeffs(stats, m_rows, g, be):
    st = stats.reshape(-1, 8, stats.shape[-1])
    ssum = jnp.sum(st[:, 0, :], axis=0)
    ssq = jnp.sum(st[:, 1, :], axis=0)
    mean = ssum / m_rows
    var = jnp.maximum(ssq / m_rows - mean * mean, 0.0)
    s = g * jax.lax.rsqrt(var + BN_EPS)
    t = be - mean * s
    return s, t


def _taps_matmul(zb, b_ref, Ho):
    NB, _, _, K4 = zb.shape
    acc = None
    for t, (di, dj) in enumerate(((0, 0), (0, 1), (1, 0), (1, 1))):
        a = zb[:, di:di + Ho, dj:dj + Ho, :].reshape(NB * Ho * Ho, K4)
        d = jnp.dot(a, b_ref[t], preferred_element_type=jnp.float32)
        acc = d if acc is None else acc + d
    return acc


def _emit_z(y4):
    nb, H, _, C = y4.shape
    Z = H // 2 + 1
    zr = jnp.zeros((nb, 1, H, C), jnp.float32)
    t = jnp.concatenate([zr, y4, zr], axis=1)
    zc = jnp.zeros((nb, H + 2, 1, C), jnp.float32)
    zp = jnp.concatenate([zc, t, zc], axis=2)
    z6 = zp.reshape(nb, Z, 2, Z, 2, C)
    planes = [z6[:, :, qi, :, qj, :] for qi in (0, 1) for qj in (0, 1)]
    return jnp.concatenate(planes, axis=-1)


def _border_mask(Z, C4, C):
    zi = jax.lax.broadcasted_iota(jnp.int32, (Z, Z, C4), 0)
    zj = jax.lax.broadcasted_iota(jnp.int32, (Z, Z, C4), 1)
    ll = jax.lax.broadcasted_iota(jnp.int32, (Z, Z, C4), 2)
    qi = ll // (2 * C)
    qj = (ll // C) % 2
    border = ((zi == 0) & (qi == 0)) | ((zi == Z - 1) & (qi == 1)) \
        | ((zj == 0) & (qj == 0)) | ((zj == Z - 1) & (qj == 1))
    return jnp.where(border, 0.0, 1.0).astype(jnp.float32)


def _l1_kernel(z_ref, b_ref, bias_ref, zo_ref, *, nb):
    y = _taps_matmul(z_ref[...], b_ref, 16) + bias_ref[...]
    y = jnp.where(y > 0, y, LEAKY_SLOPE * y)
    y4 = y.reshape(nb, 16, 16, 256)
    imgs = jnp.concatenate(
        [y4[:, :, :, m * 64:(m + 1) * 64].reshape(nb, 1, 16, 16, 64)
         for m in range(4)], axis=1).reshape(nb * 4, 16, 16, 64)
    zo_ref[...] = _emit_z(imgs)


def _conv_kernel(z_ref, b_ref, zo_ref, st_ref, *, Ho, nb):
    acc = _taps_matmul(z_ref[...], b_ref, Ho)
    st_ref[0:1, :] = jnp.sum(acc, axis=0, keepdims=True)
    st_ref[1:2, :] = jnp.sum(acc * acc, axis=0, keepdims=True)
    zo_ref[...] = _emit_z(acc.reshape(nb, Ho, Ho, acc.shape[-1]))


def _affine_conv_kernel(z_ref, b_ref, s_ref, t_ref, zo_ref, st_ref, *,
                        Ho, nb, C):
    zb = z_ref[...]
    Z = zb.shape[1]
    y = zb * s_ref[...] + t_ref[...]
    y = jnp.where(y > 0, y, LEAKY_SLOPE * y)
    y = y * _border_mask(Z, zb.shape[-1], C)
    acc = _taps_matmul(y, b_ref, Ho)
    st_ref[0:1, :] = jnp.sum(acc, axis=0, keepdims=True)
    st_ref[1:2, :] = jnp.sum(acc * acc, axis=0, keepdims=True)
    if Ho > 1:
        zo_ref[...] = _emit_z(acc.reshape(nb, Ho, Ho, acc.shape[-1]))
    else:
        zo_ref[...] = acc


def _l4_kernel(z_ref, b_ref, s_ref, t_ref, zo_ref, st_ref, *, nb):
    zb = z_ref[...]
    y = zb * s_ref[...] + t_ref[...]
    y = jnp.where(y > 0, y, LEAKY_SLOPE * y)
    y = y * _border_mask(3, 1024, 256)
    acc = _taps_matmul(y, b_ref, 2)
    st_ref[0:1, :] = jnp.sum(acc, axis=0, keepdims=True)
    st_ref[1:2, :] = jnp.sum(acc * acc, axis=0, keepdims=True)
    z5 = _emit_z(acc.reshape(nb, 2, 2, 512))
    zo_ref[...] = z5.reshape(nb, 8192)


def _l5_kernel(a_ref, b_ref, s_ref, t_ref, bias_ref, o_ref, acc_ref):
    k = pl.program_id(1)

    @pl.when(k == 0)
    def _():
        acc_ref[...] = jnp.zeros_like(acc_ref)

    z = a_ref[...] * s_ref[...] + t_ref[...]
    z = jnp.where(z > 0, z, LEAKY_SLOPE * z)
    acc_ref[...] += jnp.dot(z, b_ref[...], preferred_element_type=jnp.float32)

    @pl.when(k == pl.num_programs(1) - 1)
    def _():
        y = acc_ref[...] + bias_ref[...]
        o_ref[...] = 1.0 / (1.0 + jnp.exp(-y))


def kernel(x, w1, b1, w2, g2, be2, w3, g3, be3, w4, g4, be4, w5, b5):
    N = x.shape[0]
    G = N // 4

    xp = jnp.pad(x.astype(jnp.float32), ((0, 0), (0, 5), (1, 1), (1, 1)))
    z1 = (xp.reshape(G, 4, 8, 17, 2, 17, 2)
          .transpose(0, 3, 5, 1, 4, 6, 2)
          .reshape(G, 17, 17, 128))
    b32 = _wmat(w1, 8)
    eye4 = jnp.eye(4, dtype=jnp.float32)
    b1bd = jax.vmap(lambda b: jnp.kron(eye4, b))(b32)
    bias1 = jnp.tile(b1.astype(jnp.float32), 4).reshape(1, 256)

    nb1 = min(16, G)
    z2 = pl.pallas_call(
        functools.partial(_l1_kernel, nb=nb1),
        out_shape=jax.ShapeDtypeStruct((N, 9, 9, 256), jnp.float32),
        grid=(G // nb1,),
        in_specs=[
            pl.BlockSpec((nb1, 17, 17, 128), lambda m: (m, 0, 0, 0)),
            pl.BlockSpec((4, 128, 256), lambda m: (0, 0, 0)),
            pl.BlockSpec((1, 256), lambda m: (0, 0)),
        ],
        out_specs=pl.BlockSpec((nb1 * 4, 9, 9, 256), lambda m: (m, 0, 0, 0)),
        compiler_params=pltpu.CompilerParams(
            dimension_semantics=("parallel",)),
    )(z1, b1bd, bias1)

    nb2 = min(64, N)
    z3, st2 = pl.pallas_call(
        functools.partial(_conv_kernel, Ho=8, nb=nb2),
        out_shape=(jax.ShapeDtypeStruct((N, 5, 5, 512), jnp.float32),
                   jax.ShapeDtypeStruct((N // nb2 * 8, 128), jnp.float32)),
        grid=(N // nb2,),
        in_specs=[
            pl.BlockSpec((nb2, 9, 9, 256), lambda m: (m, 0, 0, 0)),
            pl.BlockSpec((4, 256, 128), lambda m: (0, 0, 0)),
        ],
        out_specs=(pl.BlockSpec((nb2, 5, 5, 512), lambda m: (m, 0, 0, 0)),
                   pl.BlockSpec((8, 128), lambda m: (m, 0))),
        compiler_params=pltpu.CompilerParams(
            dimension_semantics=("parallel",)),
    )(z2, _wmat(w2, 64))
    s2, t2 = _bn_coeffs(st2, N * 64, g2, be2)

    nb3 = min(128, N)
    z4, st3 = pl.pallas_call(
        functools.partial(_affine_conv_kernel, Ho=4, nb=nb3, C=128),
        out_shape=(jax.ShapeDtypeStruct((N, 3, 3, 1024), jnp.float32),
                   jax.ShapeDtypeStruct((N // nb3 * 8, 256), jnp.float32)),
        grid=(N // nb3,),
        in_specs=[
            pl.BlockSpec((nb3, 5, 5, 512), lambda m: (m, 0, 0, 0)),
            pl.BlockSpec((4, 512, 256), lambda m: (0, 0, 0)),
            pl.BlockSpec((1, 512), lambda m: (0, 0)),
            pl.BlockSpec((1, 512), lambda m: (0, 0)),
        ],
        out_specs=(pl.BlockSpec((nb3, 3, 3, 1024), lambda m: (m, 0, 0, 0)),
                   pl.BlockSpec((8, 256), lambda m: (m, 0))),
        compiler_params=pltpu.CompilerParams(
            dimension_semantics=("parallel",)),
    )(z3, _wmat(w3, 128),
      jnp.tile(s2, 4).reshape(1, 512), jnp.tile(t2, 4).reshape(1, 512))
    s3, t3 = _bn_coeffs(st3, N * 16, g3, be3)

    nb4 = min(128, N)
    z5, st4 = pl.pallas_call(
        functools.partial(_l4_kernel, nb=nb4),
        out_shape=(jax.ShapeDtypeStruct((N, 8192), jnp.float32),
                   jax.ShapeDtypeStruct((N // nb4 * 8, 512), jnp.float32)),
        grid=(N // nb4,),
        in_specs=[
            pl.BlockSpec((nb4, 3, 3, 1024), lambda m: (m, 0, 0, 0)),
            pl.BlockSpec((4, 1024, 512), lambda m: (0, 0, 0)),
            pl.BlockSpec((1, 1024), lambda m: (0, 0)),
            pl.BlockSpec((1, 1024), lambda m: (0, 0)),
        ],
        out_specs=(pl.BlockSpec((nb4, 8192), lambda m: (m, 0)),
                   pl.BlockSpec((8, 512), lambda m: (m, 0))),
        compiler_params=pltpu.CompilerParams(
            dimension_semantics=("parallel",)),
    )(z4, _wmat(w4, 256),
      jnp.tile(s3, 4).reshape(1, 1024), jnp.tile(t3, 4).reshape(1, 1024))
    s4, t4 = _bn_coeffs(st4, N * 4, g4, be4)

    b5m = jnp.pad(_wmat(w5, 512).reshape(8192, 1), ((0, 0), (0, 127)))
    bias5 = jnp.pad(b5.astype(jnp.float32), (0, 127)).reshape(1, 128)
    ll = jnp.arange(8192)
    zi5, zj5 = ll // 4096, (ll // 2048) % 2
    qi5, qj5 = (ll // 1024) % 2, (ll // 512) % 2
    live = jnp.logical_not(((zi5 == 0) & (qi5 == 0)) | ((zi5 == 1) & (qi5 == 1))
                           | ((zj5 == 0) & (qj5 == 0)) | ((zj5 == 1) & (qj5 == 1))
                           ).astype(jnp.float32)
    s4z = (jnp.tile(s4, 16) * live).reshape(1, 8192)
    t4z = (jnp.tile(t4, 16) * live).reshape(1, 8192)

    nb5 = N // 2
    y = pl.pallas_call(
        _l5_kernel,
        out_shape=jax.ShapeDtypeStruct((N, 128), jnp.float32),
        grid=(2, 4),
        in_specs=[
            pl.BlockSpec((nb5, 2048), lambda m, k: (m, k)),
            pl.BlockSpec((2048, 128), lambda m, k: (k, 0)),
            pl.BlockSpec((1, 2048), lambda m, k: (0, k)),
            pl.BlockSpec((1, 2048), lambda m, k: (0, k)),
            pl.BlockSpec((1, 128), lambda m, k: (0, 0)),
        ],
        out_specs=pl.BlockSpec((nb5, 128), lambda m, k: (m, 0)),
        scratch_shapes=[pltpu.VMEM((nb5, 128), jnp.float32)],
        compiler_params=pltpu.CompilerParams(
            dimension_semantics=("parallel", "arbitrary")),
    )(z5, b5m, s4z, t4z, bias5)

    return y[:, :1].reshape(N, 1, 1, 1)


# speedup vs baseline: 179.8196x; 3.2549x over previous
import functools

import jax
import jax.numpy as jnp
from jax.experimental import pallas as pl
from jax.experimental.pallas import tpu as pltpu

LEAKY_SLOPE = 0.2
BN_EPS = 1e-5


def _wmat(w, C):
    Co, Ci = w.shape[0], w.shape[1]
    Wt = jnp.transpose(w, (2, 3, 1, 0)).astype(jnp.float32)
    Wt = jnp.pad(Wt, ((0, 0), (0, 0), (0, C - Ci), (0, 0)))
    Wt = Wt.reshape(2, 2, 2, 2, C, Co).transpose(0, 2, 1, 3, 4, 5)
    return Wt.reshape(4, 4 * C, Co)


def _bn_coeffs(stats, m_rows, g, be):
    st = stats.reshape(-1, 8, stats.shape[-1])
    ssum = jnp.sum(st[:, 0, :], axis=0)
    ssq = jnp.sum(st[:, 1, :], axis=0)
    mean = ssum / m_rows
    var = jnp.maximum(ssq / m_rows - mean * mean, 0.0)
    s = g * jax.lax.rsqrt(var + BN_EPS)
    t = be - mean * s
    return s, t


def _taps_matmul(zb, b_ref, Ho):
    NB, _, _, K4 = zb.shape
    acc = None
    for t, (di, dj) in enumerate(((0, 0), (0, 1), (1, 0), (1, 1))):
        a = zb[:, di:di + Ho, dj:dj + Ho, :].reshape(NB * Ho * Ho, K4)
        d = jnp.dot(a, b_ref[t], preferred_element_type=jnp.float32)
        acc = d if acc is None else acc + d
    return acc


def _emit_z(y4):
    nb, H, _, C = y4.shape
    Z = H // 2 + 1
    zr = jnp.zeros((nb, 1, H, C), jnp.float32)
    t = jnp.concatenate([zr, y4, zr], axis=1)
    zc = jnp.zeros((nb, H + 2, 1, C), jnp.float32)
    zp = jnp.concatenate([zc, t, zc], axis=2)
    z6 = zp.reshape(nb, Z, 2, Z, 2, C)
    planes = [z6[:, :, qi, :, qj, :] for qi in (0, 1) for qj in (0, 1)]
    return jnp.concatenate(planes, axis=-1)


def _border_mask(Z, C4, C):
    zi = jax.lax.broadcasted_iota(jnp.int32, (Z, Z, C4), 0)
    zj = jax.lax.broadcasted_iota(jnp.int32, (Z, Z, C4), 1)
    ll = jax.lax.broadcasted_iota(jnp.int32, (Z, Z, C4), 2)
    qi = ll // (2 * C)
    qj = (ll // C) % 2
    border = ((zi == 0) & (qi == 0)) | ((zi == Z - 1) & (qi == 1)) \
        | ((zj == 0) & (qj == 0)) | ((zj == Z - 1) & (qj == 1))
    return jnp.where(border, 0.0, 1.0).astype(jnp.float32)


def _l1_kernel(xq_ref, t_ref, bias_ref, zo_ref, *, nb):
    xb = xq_ref[...]
    acc = None
    for di in (0, 1):
        for qi in (0, 1):
            a = jnp.concatenate(
                [xb[:, qi, c, di:di + 16, :] for c in range(3)],
                axis=-1).reshape(nb * 16, 102)
            d = jnp.dot(a, t_ref[2 * di + qi],
                        preferred_element_type=jnp.float32)
            acc = d if acc is None else acc + d
    y = acc + bias_ref[...]
    y = jnp.where(y > 0, y, LEAKY_SLOPE * y)
    zo_ref[...] = _emit_z(y.reshape(nb, 16, 16, 64))


def _conv_kernel(z_ref, b_ref, zo_ref, st_ref, *, Ho, nb):
    acc = _taps_matmul(z_ref[...], b_ref, Ho)
    st_ref[0:1, :] = jnp.sum(acc, axis=0, keepdims=True)
    st_ref[1:2, :] = jnp.sum(acc * acc, axis=0, keepdims=True)
    zo_ref[...] = _emit_z(acc.reshape(nb, Ho, Ho, acc.shape[-1]))


def _affine_conv_kernel(z_ref, b_ref, s_ref, t_ref, zo_ref, st_ref, *,
                        Ho, nb, C):
    zb = z_ref[...]
    Z = zb.shape[1]
    y = zb * s_ref[...] + t_ref[...]
    y = jnp.where(y > 0, y, LEAKY_SLOPE * y)
    y = y * _border_mask(Z, zb.shape[-1], C)
    acc = _taps_matmul(y, b_ref, Ho)
    st_ref[0:1, :] = jnp.sum(acc, axis=0, keepdims=True)
    st_ref[1:2, :] = jnp.sum(acc * acc, axis=0, keepdims=True)
    if Ho > 1:
        zo_ref[...] = _emit_z(acc.reshape(nb, Ho, Ho, acc.shape[-1]))
    else:
        zo_ref[...] = acc


def _l4_kernel(z_ref, b_ref, s_ref, t_ref, zo_ref, st_ref, *, nb):
    zb = z_ref[...]
    y = zb * s_ref[...] + t_ref[...]
    y = jnp.where(y > 0, y, LEAKY_SLOPE * y)
    y = y * _border_mask(3, 1024, 256)
    acc = _taps_matmul(y, b_ref, 2)
    st_ref[0:1, :] = jnp.sum(acc, axis=0, keepdims=True)
    st_ref[1:2, :] = jnp.sum(acc * acc, axis=0, keepdims=True)
    z5 = _emit_z(acc.reshape(nb, 2, 2, 512))
    zo_ref[...] = z5.reshape(nb, 8192)


def _l5_kernel(a_ref, b_ref, s_ref, t_ref, bias_ref, o_ref, acc_ref):
    k = pl.program_id(1)

    @pl.when(k == 0)
    def _():
        acc_ref[...] = jnp.zeros_like(acc_ref)

    z = a_ref[...] * s_ref[...] + t_ref[...]
    z = jnp.where(z > 0, z, LEAKY_SLOPE * z)
    acc_ref[...] += jnp.dot(z, b_ref[...], preferred_element_type=jnp.float32)

    @pl.when(k == pl.num_programs(1) - 1)
    def _():
        y = acc_ref[...] + bias_ref[...]
        o_ref[...] = 1.0 / (1.0 + jnp.exp(-y))


def kernel(x, w1, b1, w2, g2, be2, w3, g3, be3, w4, g4, be4, w5, b5):
    N = x.shape[0]

    xpad = jnp.pad(x.astype(jnp.float32), ((0, 0), (0, 0), (1, 1), (1, 1)))
    xq = xpad.reshape(N, 3, 17, 2, 34).transpose(0, 3, 1, 2, 4)
    w1f = w1.astype(jnp.float32)
    tsel = jnp.zeros((4, 3, 34, 16, 64), jnp.float32)
    ow = jnp.arange(16)
    for kw in range(4):
        upd = jnp.broadcast_to(
            jnp.transpose(w1f[:, :, :, kw], (2, 1, 0))[:, :, None, :],
            (4, 3, 16, 64))
        tsel = tsel.at[:, :, 2 * ow + kw, ow, :].add(upd)
    tsel = tsel.reshape(4, 102, 1024)
    bias1 = jnp.tile(b1.astype(jnp.float32), 16).reshape(1, 1024)

    nb1 = min(64, N)
    z2 = pl.pallas_call(
        functools.partial(_l1_kernel, nb=nb1),
        out_shape=jax.ShapeDtypeStruct((N, 9, 9, 256), jnp.float32),
        grid=(N // nb1,),
        in_specs=[
            pl.BlockSpec((nb1, 2, 3, 17, 34), lambda m: (m, 0, 0, 0, 0)),
            pl.BlockSpec((4, 102, 1024), lambda m: (0, 0, 0)),
            pl.BlockSpec((1, 1024), lambda m: (0, 0)),
        ],
        out_specs=pl.BlockSpec((nb1, 9, 9, 256), lambda m: (m, 0, 0, 0)),
        compiler_params=pltpu.CompilerParams(
            dimension_semantics=("parallel",)),
    )(xq, tsel, bias1)

    nb2 = min(64, N)
    z3, st2 = pl.pallas_call(
        functools.partial(_conv_kernel, Ho=8, nb=nb2),
        out_shape=(jax.ShapeDtypeStruct((N, 5, 5, 512), jnp.float32),
                   jax.ShapeDtypeStruct((N // nb2 * 8, 128), jnp.float32)),
        grid=(N // nb2,),
        in_specs=[
            pl.BlockSpec((nb2, 9, 9, 256), lambda m: (m, 0, 0, 0)),
            pl.BlockSpec((4, 256, 128), lambda m: (0, 0, 0)),
        ],
        out_specs=(pl.BlockSpec((nb2, 5, 5, 512), lambda m: (m, 0, 0, 0)),
                   pl.BlockSpec((8, 128), lambda m: (m, 0))),
        compiler_params=pltpu.CompilerParams(
            dimension_semantics=("parallel",)),
    )(z2, _wmat(w2, 64))
    s2, t2 = _bn_coeffs(st2, N * 64, g2, be2)

    nb3 = min(128, N)
    z4, st3 = pl.pallas_call(
        functools.partial(_affine_conv_kernel, Ho=4, nb=nb3, C=128),
        out_shape=(jax.ShapeDtypeStruct((N, 3, 3, 1024), jnp.float32),
                   jax.ShapeDtypeStruct((N // nb3 * 8, 256), jnp.float32)),
        grid=(N // nb3,),
        in_specs=[
            pl.BlockSpec((nb3, 5, 5, 512), lambda m: (m, 0, 0, 0)),
            pl.BlockSpec((4, 512, 256), lambda m: (0, 0, 0)),
            pl.BlockSpec((1, 512), lambda m: (0, 0)),
            pl.BlockSpec((1, 512), lambda m: (0, 0)),
        ],
        out_specs=(pl.BlockSpec((nb3, 3, 3, 1024), lambda m: (m, 0, 0, 0)),
                   pl.BlockSpec((8, 256), lambda m: (m, 0))),
        compiler_params=pltpu.CompilerParams(
            dimension_semantics=("parallel",)),
    )(z3, _wmat(w3, 128),
      jnp.tile(s2, 4).reshape(1, 512), jnp.tile(t2, 4).reshape(1, 512))
    s3, t3 = _bn_coeffs(st3, N * 16, g3, be3)

    nb4 = min(128, N)
    z5, st4 = pl.pallas_call(
        functools.partial(_l4_kernel, nb=nb4),
        out_shape=(jax.ShapeDtypeStruct((N, 8192), jnp.float32),
                   jax.ShapeDtypeStruct((N // nb4 * 8, 512), jnp.float32)),
        grid=(N // nb4,),
        in_specs=[
            pl.BlockSpec((nb4, 3, 3, 1024), lambda m: (m, 0, 0, 0)),
            pl.BlockSpec((4, 1024, 512), lambda m: (0, 0, 0)),
            pl.BlockSpec((1, 1024), lambda m: (0, 0)),
            pl.BlockSpec((1, 1024), lambda m: (0, 0)),
        ],
        out_specs=(pl.BlockSpec((nb4, 8192), lambda m: (m, 0)),
                   pl.BlockSpec((8, 512), lambda m: (m, 0))),
        compiler_params=pltpu.CompilerParams(
            dimension_semantics=("parallel",)),
    )(z4, _wmat(w4, 256),
      jnp.tile(s3, 4).reshape(1, 1024), jnp.tile(t3, 4).reshape(1, 1024))
    s4, t4 = _bn_coeffs(st4, N * 4, g4, be4)

    b5m = jnp.pad(_wmat(w5, 512).reshape(8192, 1), ((0, 0), (0, 127)))
    bias5 = jnp.pad(b5.astype(jnp.float32), (0, 127)).reshape(1, 128)
    ll = jnp.arange(8192)
    zi5, zj5 = ll // 4096, (ll // 2048) % 2
    qi5, qj5 = (ll // 1024) % 2, (ll // 512) % 2
    live = jnp.logical_not(((zi5 == 0) & (qi5 == 0)) | ((zi5 == 1) & (qi5 == 1))
                           | ((zj5 == 0) & (qj5 == 0)) | ((zj5 == 1) & (qj5 == 1))
                           ).astype(jnp.float32)
    s4z = (jnp.tile(s4, 16) * live).reshape(1, 8192)
    t4z = (jnp.tile(t4, 16) * live).reshape(1, 8192)

    nb5 = N // 2
    y = pl.pallas_call(
        _l5_kernel,
        out_shape=jax.ShapeDtypeStruct((N, 128), jnp.float32),
        grid=(2, 4),
        in_specs=[
            pl.BlockSpec((nb5, 2048), lambda m, k: (m, k)),
            pl.BlockSpec((2048, 128), lambda m, k: (k, 0)),
            pl.BlockSpec((1, 2048), lambda m, k: (0, k)),
            pl.BlockSpec((1, 2048), lambda m, k: (0, k)),
            pl.BlockSpec((1, 128), lambda m, k: (0, 0)),
        ],
        out_specs=pl.BlockSpec((nb5, 128), lambda m, k: (m, 0)),
        scratch_shapes=[pltpu.VMEM((nb5, 128), jnp.float32)],
        compiler_params=pltpu.CompilerParams(
            dimension_semantics=("parallel", "arbitrary")),
    )(z5, b5m, s4z, t4z, bias5)

    return y[:, :1].reshape(N, 1, 1, 1)
